# baseline (speedup 1.0000x reference)
import functools

import jax
import jax.numpy as jnp
from jax import lax
from jax.experimental import pallas as pl
from jax.experimental.pallas import tpu as pltpu
from jax.experimental.pallas import tpu_sc as plsc

B, C, H, W = 8, 768, 32, 32
N_EMBED, EMBED_DIM = 8192, 256
T = H * W
CT = 1024
NCT = N_EMBED // CT
N_TOK = B * T

_SC_CORES = 2
_SC_SUBCORES = 16
_SC_WORKERS = _SC_CORES * _SC_SUBCORES
_TOK_PER_WORKER = N_TOK // _SC_WORKERS


def _vq_body(z_ref, pwt_ref, pb_ref, gm_ref, bt_ref, mu_ref, var_ref, et_ref,
             etb_ref, ind_ref, dsum_ref, zeb_ref, f2_ref, rmin_ref, rarg_ref,
             e2_ref):
    b = pl.program_id(0)
    j = pl.program_id(1)

    @pl.when(j == 0)
    def _init():
        zb = z_ref[0]
        t = jnp.dot(zb.astype(jnp.bfloat16), pwt_ref[...].astype(jnp.bfloat16),
                    preferred_element_type=jnp.float32)
        t = t + pb_ref[...]
        t = (t - mu_ref[...]) / jnp.sqrt(var_ref[...] + 1e-5)
        t = t * gm_ref[...] + bt_ref[...]
        zeb_ref[...] = t.astype(jnp.bfloat16)
        f2_ref[...] = jnp.sum(t * t, axis=1, keepdims=True)
        rmin_ref[...] = jnp.full((T, 1), jnp.inf, jnp.float32)
        rarg_ref[...] = jnp.zeros((T, 1), jnp.int32)

    @pl.when(b == 0)
    def _e2_init():
        et = et_ref[...]
        e2_ref[pl.ds(j, 1), :] = jnp.sum(et * et, axis=0, keepdims=True)

    e2 = e2_ref[pl.ds(j, 1), :]
    s = jnp.dot(zeb_ref[...], etb_ref[...],
                preferred_element_type=jnp.float32)
    d = (f2_ref[...] - 2.0 * s) + e2
    tmin = jnp.min(d, axis=1, keepdims=True)
    col = lax.broadcasted_iota(jnp.int32, (T, CT), 1)
    targ = jnp.min(jnp.where(d == tmin, col, jnp.int32(2**31 - 1)),
                   axis=1, keepdims=True) + j * CT
    upd = tmin < rmin_ref[...]
    rmin_ref[...] = jnp.where(upd, tmin, rmin_ref[...])
    rarg_ref[...] = jnp.where(upd, targ, rarg_ref[...])

    @pl.when(j == NCT // 2 - 1)
    def _carry_round():
        rmin_ref[...] = rmin_ref[...].astype(jnp.bfloat16).astype(jnp.float32)

    @pl.when(j == NCT - 1)
    def _fin():
        ind_ref[...] = rarg_ref[...].reshape(1, T, 1)
        dsum_ref[b, 0] = jnp.sum(rmin_ref[...])


_VQ_SPECS = dict(
    grid=(B, NCT),
    in_specs=[
        pl.BlockSpec((1, T, C), lambda b, j: (b, 0, 0)),
        pl.BlockSpec((C, EMBED_DIM), lambda b, j: (0, 0)),
        pl.BlockSpec((1, EMBED_DIM), lambda b, j: (0, 0)),
        pl.BlockSpec((1, EMBED_DIM), lambda b, j: (0, 0)),
        pl.BlockSpec((1, EMBED_DIM), lambda b, j: (0, 0)),
        pl.BlockSpec((1, EMBED_DIM), lambda b, j: (0, 0)),
        pl.BlockSpec((1, EMBED_DIM), lambda b, j: (0, 0)),
        pl.BlockSpec((EMBED_DIM, CT), lambda b, j: (0, j)),
        pl.BlockSpec((EMBED_DIM, CT), lambda b, j: (0, j)),
    ],
    out_specs=[
        pl.BlockSpec((1, T, 1), lambda b, j: (b, 0, 0)),
        pl.BlockSpec((B, 1), lambda b, j: (0, 0), memory_space=pltpu.SMEM),
    ],
    out_shape=[
        jax.ShapeDtypeStruct((B, T, 1), jnp.int32),
        jax.ShapeDtypeStruct((B, 1), jnp.float32),
    ],
    scratch_shapes=[
        pltpu.VMEM((T, EMBED_DIM), jnp.bfloat16),
        pltpu.VMEM((T, 1), jnp.float32),
        pltpu.VMEM((T, 1), jnp.float32),
        pltpu.VMEM((T, 1), jnp.int32),
        pltpu.VMEM((NCT, CT), jnp.float32),
    ],
)


def _gather_body(table_hbm, idx_hbm, out_hbm, idx_v, rows_v, sem):
    wid = lax.axis_index("s") * _SC_CORES + lax.axis_index("c")
    base = wid * _TOK_PER_WORKER
    pltpu.sync_copy(idx_hbm.at[pl.ds(base, _TOK_PER_WORKER)], idx_v)
    pltpu.async_copy(table_hbm.at[idx_v], rows_v, sem).wait()
    pltpu.sync_copy(rows_v, out_hbm.at[pl.ds(base, _TOK_PER_WORKER)])


@functools.cache
def _sc_gather():
    return pl.kernel(
        _gather_body,
        out_type=jax.ShapeDtypeStruct((N_TOK, EMBED_DIM), jnp.float32),
        mesh=plsc.VectorSubcoreMesh(core_axis_name="c", subcore_axis_name="s"),
        scratch_types=[
            pltpu.VMEM((_TOK_PER_WORKER,), jnp.int32),
            pltpu.VMEM((_TOK_PER_WORKER, EMBED_DIM), jnp.float32),
            pltpu.SemaphoreType.DMA,
        ],
    )


def kernel(z, proj_w, proj_b, bn_gamma, bn_beta, bn_mean, bn_var, embed_w):
    zt = z.reshape(B, C, T).transpose(0, 2, 1)
    row = lambda v: v.reshape(1, EMBED_DIM)
    embT = embed_w.T
    ind3, dsum = pl.pallas_call(_vq_body, **_VQ_SPECS)(
        zt, proj_w.T, row(proj_b), row(bn_gamma), row(bn_beta),
        row(bn_mean), row(bn_var), embT, embT.astype(jnp.bfloat16))
    ind_flat = ind3.reshape(N_TOK)
    z_q = _sc_gather()(embed_w, ind_flat)
    z_q_out = z_q.reshape(B, T, EMBED_DIM).transpose(0, 2, 1).reshape(B, EMBED_DIM, H, W)
    mse = jnp.sum(dsum) / jnp.float32(N_TOK * EMBED_DIM)
    diff = (0.25 * mse + mse) * 10.0
    ind = ind3.reshape(B, H, W)
    return (z_q_out, diff, ind)


# speedup vs baseline: 1.1530x; 1.1530x over previous
import functools

import jax
import jax.numpy as jnp
from jax import lax
from jax.experimental import pallas as pl
from jax.experimental.pallas import tpu as pltpu
from jax.experimental.pallas import tpu_sc as plsc

B, C, H, W = 8, 768, 32, 32
N_EMBED, EMBED_DIM = 8192, 256
T = H * W
CT = 2048
NCT = N_EMBED // CT
N_TOK = B * T

_SC_CORES = 2
_SC_SUBCORES = 16
_SC_WORKERS = _SC_CORES * _SC_SUBCORES
_TOK_PER_WORKER = N_TOK // _SC_WORKERS


def _vq_body(z_ref, pwt_ref, pb_ref, gm_ref, bt_ref, mu_ref, var_ref, et_ref,
             ind_ref, dsum_ref, zeb_ref, f2_ref, rmin_ref, rarg_ref):
    b = pl.program_id(0)
    j = pl.program_id(1)

    @pl.when(j == 0)
    def _init():
        zb = z_ref[0]
        t = jnp.dot(zb.astype(jnp.bfloat16), pwt_ref[...].astype(jnp.bfloat16),
                    preferred_element_type=jnp.float32)
        t = t + pb_ref[...]
        t = (t - mu_ref[...]) / jnp.sqrt(var_ref[...] + 1e-5)
        t = t * gm_ref[...] + bt_ref[...]
        zeb_ref[...] = (t + t).astype(jnp.bfloat16)
        f2_ref[...] = jnp.sum(t * t, axis=1, keepdims=True)
        rmin_ref[...] = jnp.full((T, 1), jnp.inf, jnp.float32)
        rarg_ref[...] = jnp.zeros((T, 1), jnp.int32)

    et = et_ref[...]
    e2 = jnp.sum(et * et, axis=0, keepdims=True)
    s = jnp.dot(zeb_ref[...], et.astype(jnp.bfloat16),
                preferred_element_type=jnp.float32)
    d = (f2_ref[...] - s) + e2
    tmin = jnp.min(d, axis=1, keepdims=True)
    col = lax.broadcasted_iota(jnp.int32, (T, CT), 1)
    targ = jnp.min(jnp.where(d == tmin, col, jnp.int32(2**31 - 1)),
                   axis=1, keepdims=True) + j * CT
    upd = tmin < rmin_ref[...]
    rmin_ref[...] = jnp.where(upd, tmin, rmin_ref[...])
    rarg_ref[...] = jnp.where(upd, targ, rarg_ref[...])

    @pl.when(j == NCT // 2 - 1)
    def _carry_round():
        rmin_ref[...] = rmin_ref[...].astype(jnp.bfloat16).astype(jnp.float32)

    @pl.when(j == NCT - 1)
    def _fin():
        ind_ref[...] = rarg_ref[...].reshape(1, T, 1)
        dsum_ref[b, 0] = jnp.sum(rmin_ref[...])


_VQ_SPECS = dict(
    grid=(B, NCT),
    in_specs=[
        pl.BlockSpec((1, T, C), lambda b, j: (b, 0, 0)),
        pl.BlockSpec((C, EMBED_DIM), lambda b, j: (0, 0)),
        pl.BlockSpec((1, EMBED_DIM), lambda b, j: (0, 0)),
        pl.BlockSpec((1, EMBED_DIM), lambda b, j: (0, 0)),
        pl.BlockSpec((1, EMBED_DIM), lambda b, j: (0, 0)),
        pl.BlockSpec((1, EMBED_DIM), lambda b, j: (0, 0)),
        pl.BlockSpec((1, EMBED_DIM), lambda b, j: (0, 0)),
        pl.BlockSpec((EMBED_DIM, CT), lambda b, j: (0, j)),
    ],
    out_specs=[
        pl.BlockSpec((1, T, 1), lambda b, j: (b, 0, 0)),
        pl.BlockSpec((B, 1), lambda b, j: (0, 0), memory_space=pltpu.SMEM),
    ],
    out_shape=[
        jax.ShapeDtypeStruct((B, T, 1), jnp.int32),
        jax.ShapeDtypeStruct((B, 1), jnp.float32),
    ],
    scratch_shapes=[
        pltpu.VMEM((T, EMBED_DIM), jnp.bfloat16),
        pltpu.VMEM((T, 1), jnp.float32),
        pltpu.VMEM((T, 1), jnp.float32),
        pltpu.VMEM((T, 1), jnp.int32),
    ],
)


def _gather_body(table_hbm, idx_hbm, out_hbm, idx_v, rows_v, sem):
    wid = lax.axis_index("s") * _SC_CORES + lax.axis_index("c")
    base = wid * _TOK_PER_WORKER
    pltpu.sync_copy(idx_hbm.at[pl.ds(base, _TOK_PER_WORKER)], idx_v)
    pltpu.async_copy(table_hbm.at[idx_v], rows_v, sem).wait()
    pltpu.sync_copy(rows_v, out_hbm.at[pl.ds(base, _TOK_PER_WORKER)])


@functools.cache
def _sc_gather():
    return pl.kernel(
        _gather_body,
        out_type=jax.ShapeDtypeStruct((N_TOK, EMBED_DIM), jnp.float32),
        mesh=plsc.VectorSubcoreMesh(core_axis_name="c", subcore_axis_name="s"),
        scratch_types=[
            pltpu.VMEM((_TOK_PER_WORKER,), jnp.int32),
            pltpu.VMEM((_TOK_PER_WORKER, EMBED_DIM), jnp.float32),
            pltpu.SemaphoreType.DMA,
        ],
    )


def kernel(z, proj_w, proj_b, bn_gamma, bn_beta, bn_mean, bn_var, embed_w):
    zt = z.reshape(B, C, T).transpose(0, 2, 1)
    row = lambda v: v.reshape(1, EMBED_DIM)
    ind3, dsum = pl.pallas_call(_vq_body, **_VQ_SPECS)(
        zt, proj_w.T, row(proj_b), row(bn_gamma), row(bn_beta),
        row(bn_mean), row(bn_var), embed_w.T)
    ind_flat = ind3.reshape(N_TOK)
    z_q = _sc_gather()(embed_w, ind_flat)
    z_q_out = z_q.reshape(B, T, EMBED_DIM).transpose(0, 2, 1).reshape(B, EMBED_DIM, H, W)
    mse = jnp.sum(dsum) / jnp.float32(N_TOK * EMBED_DIM)
    diff = (0.25 * mse + mse) * 10.0
    ind = ind3.reshape(B, H, W)
    return (z_q_out, diff, ind)


# speedup vs baseline: 1.2109x; 1.0502x over previous
import functools

import jax
import jax.numpy as jnp
from jax import lax
from jax.experimental import pallas as pl
from jax.experimental.pallas import tpu as pltpu
from jax.experimental.pallas import tpu_sc as plsc

B, C, H, W = 8, 768, 32, 32
N_EMBED, EMBED_DIM = 8192, 256
T = H * W
CT = 4096
NCT = N_EMBED // CT
N_TOK = B * T

_SC_CORES = 2
_SC_SUBCORES = 16
_SC_WORKERS = _SC_CORES * _SC_SUBCORES
_TOK_PER_WORKER = N_TOK // _SC_WORKERS


def _vq_body(z_ref, pwt_ref, pb_ref, gm_ref, bt_ref, mu_ref, var_ref, et_ref,
             ind_ref, dsum_ref, zeb_ref, f2_ref, rmin_ref, rarg_ref):
    b = pl.program_id(0)
    j = pl.program_id(1)

    @pl.when(j == 0)
    def _init():
        zb = z_ref[0]
        t = jnp.dot(zb.astype(jnp.bfloat16), pwt_ref[...].astype(jnp.bfloat16),
                    preferred_element_type=jnp.float32)
        t = t + pb_ref[...]
        t = (t - mu_ref[...]) / jnp.sqrt(var_ref[...] + 1e-5)
        t = t * gm_ref[...] + bt_ref[...]
        zeb_ref[...] = (t + t).astype(jnp.bfloat16)
        f2_ref[...] = jnp.sum(t * t, axis=1, keepdims=True)
        rmin_ref[...] = jnp.full((T, 1), jnp.inf, jnp.float32)
        rarg_ref[...] = jnp.zeros((T, 1), jnp.int32)

    et = et_ref[...]
    e2 = jnp.sum(et * et, axis=0, keepdims=True)
    s = jnp.dot(zeb_ref[...], et.astype(jnp.bfloat16),
                preferred_element_type=jnp.float32)
    d = (f2_ref[...] - s) + e2
    tmin = jnp.min(d, axis=1, keepdims=True)
    col = lax.broadcasted_iota(jnp.int32, (T, CT), 1)
    targ = jnp.min(jnp.where(d == tmin, col, jnp.int32(2**31 - 1)),
                   axis=1, keepdims=True) + j * CT
    upd = tmin < rmin_ref[...]
    rmin_ref[...] = jnp.where(upd, tmin, rmin_ref[...])
    rarg_ref[...] = jnp.where(upd, targ, rarg_ref[...])

    @pl.when(j == NCT // 2 - 1)
    def _carry_round():
        rmin_ref[...] = rmin_ref[...].astype(jnp.bfloat16).astype(jnp.float32)

    @pl.when(j == NCT - 1)
    def _fin():
        ind_ref[...] = rarg_ref[...].reshape(1, T, 1)
        dsum_ref[b, 0] = jnp.sum(rmin_ref[...])


_VQ_SPECS = dict(
    grid=(B, NCT),
    in_specs=[
        pl.BlockSpec((1, T, C), lambda b, j: (b, 0, 0)),
        pl.BlockSpec((C, EMBED_DIM), lambda b, j: (0, 0)),
        pl.BlockSpec((1, EMBED_DIM), lambda b, j: (0, 0)),
        pl.BlockSpec((1, EMBED_DIM), lambda b, j: (0, 0)),
        pl.BlockSpec((1, EMBED_DIM), lambda b, j: (0, 0)),
        pl.BlockSpec((1, EMBED_DIM), lambda b, j: (0, 0)),
        pl.BlockSpec((1, EMBED_DIM), lambda b, j: (0, 0)),
        pl.BlockSpec((EMBED_DIM, CT), lambda b, j: (0, j)),
    ],
    out_specs=[
        pl.BlockSpec((1, T, 1), lambda b, j: (b, 0, 0)),
        pl.BlockSpec((B, 1), lambda b, j: (0, 0), memory_space=pltpu.SMEM),
    ],
    out_shape=[
        jax.ShapeDtypeStruct((B, T, 1), jnp.int32),
        jax.ShapeDtypeStruct((B, 1), jnp.float32),
    ],
    scratch_shapes=[
        pltpu.VMEM((T, EMBED_DIM), jnp.bfloat16),
        pltpu.VMEM((T, 1), jnp.float32),
        pltpu.VMEM((T, 1), jnp.float32),
        pltpu.VMEM((T, 1), jnp.int32),
    ],
)


def _gather_body(table_hbm, idx_hbm, out_hbm, idx_v, rows_v, sem):
    wid = lax.axis_index("s") * _SC_CORES + lax.axis_index("c")
    base = wid * _TOK_PER_WORKER
    pltpu.sync_copy(idx_hbm.at[pl.ds(base, _TOK_PER_WORKER)], idx_v)
    pltpu.async_copy(table_hbm.at[idx_v], rows_v, sem).wait()
    pltpu.sync_copy(rows_v, out_hbm.at[pl.ds(base, _TOK_PER_WORKER)])


@functools.cache
def _sc_gather():
    return pl.kernel(
        _gather_body,
        out_type=jax.ShapeDtypeStruct((N_TOK, EMBED_DIM), jnp.float32),
        mesh=plsc.VectorSubcoreMesh(core_axis_name="c", subcore_axis_name="s"),
        scratch_types=[
            pltpu.VMEM((_TOK_PER_WORKER,), jnp.int32),
            pltpu.VMEM((_TOK_PER_WORKER, EMBED_DIM), jnp.float32),
            pltpu.SemaphoreType.DMA,
        ],
    )


def kernel(z, proj_w, proj_b, bn_gamma, bn_beta, bn_mean, bn_var, embed_w):
    zt = z.reshape(B, C, T).transpose(0, 2, 1)
    row = lambda v: v.reshape(1, EMBED_DIM)
    ind3, dsum = pl.pallas_call(_vq_body, **_VQ_SPECS)(
        zt, proj_w.T, row(proj_b), row(bn_gamma), row(bn_beta),
        row(bn_mean), row(bn_var), embed_w.T)
    ind_flat = ind3.reshape(N_TOK)
    z_q = _sc_gather()(embed_w, ind_flat)
    z_q_out = z_q.reshape(B, T, EMBED_DIM).transpose(0, 2, 1).reshape(B, EMBED_DIM, H, W)
    mse = jnp.sum(dsum) / jnp.float32(N_TOK * EMBED_DIM)
    diff = (0.25 * mse + mse) * 10.0
    ind = ind3.reshape(B, H, W)
    return (z_q_out, diff, ind)
